# Initial kernel scaffold; baseline (speedup 1.0000x reference)
import jax
import jax.numpy as jnp
from jax.experimental import pallas as pl


def kernel(x, edge_idx, W1, b1, W2, b2, Wd, bd):
    raise NotImplementedError("write your pallas kernel here")


# speedup vs baseline: 8.3534x; 8.3534x over previous
import functools

import jax
import jax.numpy as jnp
from jax import lax
from jax.experimental import pallas as pl
from jax.experimental.pallas import tpu as pltpu
from jax.experimental.pallas import tpu_sc as plsc

N = 10000
TROWS = 632
NPAD = 16 * TROWS
DTR = 640
DNP = 16 * DTR
MB = 1000
GRID = N // MB
EB = 128
NTILES = 16
NSC = 2

_f32 = jnp.float32


def _dis_from_degp(d_ref):
    deg = jnp.sum(d_ref[...], axis=0) + 1.0
    return lax.rsqrt(deg)


def _leaky(z):
    return jnp.where(z >= 0, z, 0.01 * z)


def _tc_pre_body(x_ref, wd_ref, bd_ref, dh_ref,
                 x0_ref, x1_ref, iden_ref):
    dis = _dis_from_degp(dh_ref)
    xb = x_ref[...]
    xs = xb * dis
    x0_ref[...] = xs[:, 0:128]
    x1_ref[...] = xs[:, 128:256]
    iden_ref[...] = jnp.dot(xb, wd_ref[...], preferred_element_type=_f32) + bd_ref[...]


def _tc_pre(x, Wd, bd, dh):
    blk = lambda shape, imap: pl.BlockSpec(shape, imap)
    row = lambda i: (i, 0)
    full = lambda i: (0, 0)
    out_rc = jax.ShapeDtypeStruct((N, 128), _f32)
    return pl.pallas_call(
        _tc_pre_body,
        grid=(GRID,),
        in_specs=[
            blk((MB, 256), row),
            blk((256, 256), full),
            blk((1, 256), full),
            pl.BlockSpec((NSC * NTILES, MB, 1), lambda i: (0, i, 0)),
        ],
        out_specs=[blk((MB, 128), row)] * 2 + [blk((MB, 256), row)],
        out_shape=[out_rc] * 2 + [jax.ShapeDtypeStruct((N, 256), _f32)],
    )(x, Wd, bd, dh)


def _tc_mid_body(s0, s1, x0, x1, m_ref, dh_ref, b1_ref, w1_ref, w2_ref,
                 o0_ref, o1_ref):
    dis = _dis_from_degp(dh_ref)
    u0 = s0[...] + x0[...]
    u1 = s1[...] + x1[...]
    z = (jnp.dot(u0, w1_ref[0:128, :], preferred_element_type=_f32)
         + jnp.dot(u1, w1_ref[128:256, :], preferred_element_type=_f32))
    z = z * dis + b1_ref[...]
    a = _leaky(z) * m_ref[...]
    hs2 = jnp.dot(a, w2_ref[...], preferred_element_type=_f32) * dis
    o0_ref[...] = hs2[:, 0:128]
    o1_ref[...] = hs2[:, 128:256]


def _tc_mid(scat1, xs, mask, dh, b1, W1, W2):
    blk = lambda shape, imap: pl.BlockSpec(shape, imap)
    row = lambda i: (i, 0)
    full = lambda i: (0, 0)
    rc = blk((MB, 128), row)
    out_rc = jax.ShapeDtypeStruct((N, 128), _f32)
    return pl.pallas_call(
        _tc_mid_body,
        grid=(GRID,),
        in_specs=[rc] * 4 + [
            blk((MB, 512), row),
            pl.BlockSpec((NSC * NTILES, MB, 1), lambda i: (0, i, 0)),
            blk((1, 512), full),
            blk((256, 512), full),
            blk((512, 256), full),
        ],
        out_specs=[rc, rc],
        out_shape=[out_rc, out_rc],
    )(*scat1, *xs, mask, dh, b1, W1, W2)


def _tc_fin_body(s0, s1, h0, h1, dh_ref, b2_ref, iden_ref, out_ref):
    dis = _dis_from_degp(dh_ref)
    outs = []
    for c, (sc, hc) in enumerate(((s0, h0), (s1, h1))):
        z = (sc[...] + hc[...]) * dis + b2_ref[:, c * 128:(c + 1) * 128]
        outs.append(_leaky(z))
    out_ref[...] = jnp.concatenate(outs, axis=1) + iden_ref[...]


def _tc_fin(scat2, hs2, dh, b2, iden):
    blk = lambda shape, imap: pl.BlockSpec(shape, imap)
    row = lambda i: (i, 0)
    full = lambda i: (0, 0)
    rc = blk((MB, 128), row)
    return pl.pallas_call(
        _tc_fin_body,
        grid=(GRID,),
        in_specs=[rc, rc, rc, rc,
                  pl.BlockSpec((NSC * NTILES, MB, 1), lambda i: (0, i, 0)),
                  blk((1, 256), full),
                  blk((MB, 256), row)],
        out_specs=blk((MB, 256), row),
        out_shape=jax.ShapeDtypeStruct((N, 256), _f32),
    )(*scat2, *hs2, dh, b2, iden)


@functools.cache
def _sc_mesh():
    return plsc.VectorSubcoreMesh(core_axis_name="c", subcore_axis_name="s",
                                  num_cores=NSC, num_subcores=NTILES)


def _deg_body(dstb, out, dstv, hist):
    cid = lax.axis_index("c")
    sid = lax.axis_index("s")
    wid = cid * NTILES + sid
    nrows = dstb.shape[0] // (NSC * NTILES)
    pltpu.sync_copy(dstb.at[pl.ds(wid * nrows, nrows)], dstv)
    zero16 = jnp.zeros((16,), _f32)
    one16 = jnp.ones((16,), _f32)

    def zbody(j, c):
        hist[pl.ds(j * 16, 16)] = zero16
        return c

    lax.fori_loop(0, DNP // 16, zbody, 0)

    def sbody(j, c):
        for k in range(EB // 16):
            idx = dstv[j, pl.ds(k * 16, 16)]
            plsc.addupdate_scatter(hist, [idx], one16)
        return c

    lax.fori_loop(0, nrows, sbody, 0)
    pltpu.sync_copy(hist, out.at[wid, 0])


def _sc_degree(dstb):
    kfn = pl.kernel(
        _deg_body,
        out_type=jax.ShapeDtypeStruct((NSC * NTILES, 1, DNP), _f32),
        mesh=_sc_mesh(),
        compiler_params=pltpu.CompilerParams(needs_layout_passes=False),
        scratch_types=[
            pltpu.VMEM((dstb.shape[0] // (NSC * NTILES), EB), jnp.int32),
            pltpu.VMEM((DNP,), _f32),
        ],
    )
    return kfn(dstb)


def _agg_body(hA, hB, srcb, dstb, zeros_in, outA, outB,
              srcv, dstv, buf0, buf1, acc, semA, semB):
    cid = lax.axis_index("c")
    sid = lax.axis_index("s")
    nrows = srcb.shape[0] // NTILES
    nhalf = nrows // 2
    pltpu.sync_copy(zeros_in, acc.at[pl.ds(sid * TROWS, TROWS)])
    plsc.subcore_barrier()

    def run(h_ref):
        def half(hf, carry):
            base = sid * nrows + hf * nhalf
            pltpu.sync_copy(srcb.at[pl.ds(base, nhalf)], srcv)
            pltpu.sync_copy(dstb.at[pl.ds(base, nhalf)], dstv)
            pltpu.async_copy(h_ref.at[srcv.at[0]], buf0, semA)

            def body(i, carry2):
                j = i * 2
                pltpu.async_copy(h_ref.at[srcv.at[j + 1]], buf1, semB)
                pltpu.make_async_copy(h_ref.at[srcv.at[j]], buf0, semA).wait()
                pltpu.sync_copy(buf0, acc.at[dstv.at[j]], add=True)

                @pl.when(i < nhalf // 2 - 1)
                def _():
                    pltpu.async_copy(h_ref.at[srcv.at[j + 2]], buf0, semA)

                pltpu.make_async_copy(h_ref.at[srcv.at[j + 1]], buf1, semB).wait()
                pltpu.sync_copy(buf1, acc.at[dstv.at[j + 1]], add=True)
                return carry2

            lax.fori_loop(0, nhalf // 2, body, 0)
            return carry

        lax.fori_loop(0, 2, half, 0)

    @pl.when(cid == 0)
    def _():
        run(hA)

    @pl.when(cid == 1)
    def _():
        run(hB)

    plsc.subcore_barrier()

    @pl.when(cid == 0)
    def _():
        pltpu.sync_copy(acc.at[pl.ds(sid * TROWS, TROWS)],
                        outA.at[pl.ds(sid * TROWS, TROWS)])

    @pl.when(cid == 1)
    def _():
        pltpu.sync_copy(acc.at[pl.ds(sid * TROWS, TROWS)],
                        outB.at[pl.ds(sid * TROWS, TROWS)])


def _sc_agg_pair(hA, hB, srcb, dstb, zeros128):
    nrows = srcb.shape[0] // NTILES
    kfn = pl.kernel(
        _agg_body,
        out_type=[jax.ShapeDtypeStruct((NPAD, 128), _f32)] * 2,
        mesh=_sc_mesh(),
        scratch_types=[
            pltpu.VMEM((nrows // 2, EB), jnp.int32),
            pltpu.VMEM((nrows // 2, EB), jnp.int32),
            pltpu.VMEM((EB, 128), _f32),
            pltpu.VMEM((EB, 128), _f32),
            pltpu.VMEM_SHARED((NPAD, 128), _f32),
            pltpu.SemaphoreType.DMA,
            pltpu.SemaphoreType.DMA,
        ],
    )
    return kfn(hA, hB, srcb, dstb, zeros128)


_MASK = None


def _dropout_mask():
    global _MASK
    if _MASK is None:
        keep = jax.random.bernoulli(jax.random.key(42), 0.5, (N, 512))
        _MASK = jnp.where(keep, _f32(2.0), _f32(0.0))
    return _MASK


def kernel(x, edge_idx, W1, b1, W2, b2, Wd, bd):
    E = edge_idx.shape[1]
    group = NSC * NTILES * EB
    EP = ((E + group - 1) // group) * group
    src = jnp.concatenate([edge_idx[0], jnp.zeros((EP - E,), jnp.int32)])
    dst = jnp.concatenate([edge_idx[1], jnp.full((EP - E,), N, jnp.int32)])
    srcb = src.reshape(EP // EB, EB)
    dstb = dst.reshape(EP // EB, EB)

    zeros128 = jnp.zeros((TROWS, 128), _f32)
    b1r = b1.reshape(1, 512)
    b2r = b2.reshape(1, 256)
    bdr = bd.reshape(1, 256)

    dh = _sc_degree(dstb).reshape(NSC * NTILES, DNP, 1)

    x0, x1, iden = _tc_pre(x, Wd, bdr, dh)

    s0, s1 = _sc_agg_pair(x0, x1, srcb, dstb, zeros128)

    o0, o1 = _tc_mid((s0, s1), (x0, x1), _dropout_mask(), dh, b1r, W1, W2)

    t0, t1 = _sc_agg_pair(o0, o1, srcb, dstb, zeros128)

    return _tc_fin((t0, t1), (o0, o1), dh, b2r, iden)
